# Initial kernel scaffold; baseline (speedup 1.0000x reference)
import jax
import jax.numpy as jnp
from jax.experimental import pallas as pl


def kernel(features, edge_index, pseudo, weight, bias):
    raise NotImplementedError("write your pallas kernel here")


# speedup vs baseline: 1.1862x; 1.1862x over previous
import dataclasses
import functools

import jax
import jax.numpy as jnp
from jax import lax
from jax.experimental import pallas as pl
from jax.experimental.pallas import tpu as pltpu
from jax.experimental.pallas import tpu_sc as plsc

N = 10000
E = 320000
F = 128
K = 25
KS = 5

NPAD = 10240
NB = 40
CH_E = 32
NTILES = 32
CHUNKS = 313
EPT = CH_E * CHUNKS
EPAD = EPT * NTILES
ROWS = EPAD // CH_E
NAGG = 10240
NPS = NAGG // 16
DROWS = NAGG // 128


def _mm_body(f_ref, w_ref, o_ref):
    o_ref[...] = jnp.dot(f_ref[...], w_ref[...],
                         preferred_element_type=jnp.float32)


def _norm_body(p_ref, d_ref, b_ref, o_ref):
    msg = p_ref[0] + p_ref[1]
    deg = d_ref[0] + d_ref[1]
    o_ref[...] = msg / jnp.maximum(deg, 1.0) + b_ref[...]


def _sc_edge_kernel(table, meta, zeros, out, degs, meta_v, dst_v, idx_v,
                    rows_v, msg_v, deg_v, ridx_v, agg_sh, deg_sh):
    cid = lax.axis_index("c")
    sid = lax.axis_index("s")
    w = sid * 2 + cid
    base_row = w * CHUNKS

    lane = lax.iota(jnp.int32, 16)

    pltpu.sync_copy(zeros, agg_sh.at[pl.ds(sid * NPS, NPS)])
    pltpu.sync_copy(zeros.at[pl.ds(0, DROWS)], deg_v)

    @pl.when(sid == 0)
    def _():
        pltpu.sync_copy(zeros.at[pl.ds(0, DROWS)], deg_sh)

    for g in range(DROWS // 16):
        ridx_v[pl.ds(16 * g, 16)] = lane + 16 * g
    plsc.subcore_barrier()

    @pl.loop(0, CHUNKS)
    def _(b):
        row = base_row + b
        pltpu.sync_copy(meta.at[row], meta_v)
        dst_v[pl.ds(0, 16)] = meta_v[pl.ds(32, 16)]
        dst_v[pl.ds(16, 16)] = meta_v[pl.ds(48, 16)]

        wregs = []
        masks = []
        dvecs = []
        for h in range(2):
            src_h = meta_v[pl.ds(16 * h, 16)]
            dvecs.append(meta_v[pl.ds(32 + 16 * h, 16)])
            wd = []
            idd = []
            for d in range(2):
                p = plsc.bitcast(meta_v[pl.ds(64 + 32 * d + 16 * h, 16)],
                                 jnp.float32)
                v = jnp.clip(p * (KS - 1), 0.0, KS - 1 - 1e-6)
                i0 = v.astype(jnp.int32)
                fr = v - i0.astype(jnp.float32)
                i1 = jnp.minimum(i0 + 1, KS - 1)
                wd.append((1.0 - fr, fr))
                idd.append((i0, i1))
            eid = (w * EPT + b * CH_E + 16 * h) + lane
            m = jnp.where(eid < E, 1.0, 0.0).astype(jnp.float32)
            masks.append(m)
            ws_h = []
            for s in range(4):
                b0 = s & 1
                b1 = (s >> 1) & 1
                ws = wd[0][b0] * wd[1][b1] * m
                ki = idd[0][b0] * KS + idd[1][b1]
                gidx = src_h * K + ki
                plsc.store_scatter(idx_v, [lane * 4 + (64 * h + s)], gidx)
                ws_h.append(ws)
            wregs.append(ws_h)

        pltpu.sync_copy(table.at[idx_v], rows_v)

        for e in range(CH_E):
            h = e // 16
            le = e % 16
            w0 = wregs[h][0][le]
            w1 = wregs[h][1][le]
            w2 = wregs[h][2][le]
            w3 = wregs[h][3][le]
            for v in range(F // 16):
                sl = pl.ds(16 * v, 16)
                acc = (rows_v[4 * e + 0, sl] * w0
                       + rows_v[4 * e + 1, sl] * w1
                       + rows_v[4 * e + 2, sl] * w2
                       + rows_v[4 * e + 3, sl] * w3)
                msg_v[e, sl] = acc
            d = dvecs[h][le]
            dr = lax.shift_right_logical(d, 7)
            dbase = lax.bitwise_and(d, 0x70)
            dlane = lax.bitwise_and(d, 0xF)
            sl_d = pl.ds(dbase, 16)
            deg_v[dr, sl_d] = deg_v[dr, sl_d] + jnp.where(
                lane == dlane, masks[h][le], 0.0)

        pltpu.sync_copy(msg_v, agg_sh.at[dst_v], add=True)

    plsc.subcore_barrier()
    pltpu.sync_copy(deg_v, deg_sh.at[ridx_v], add=True)
    plsc.subcore_barrier()

    pltpu.sync_copy(agg_sh.at[pl.ds(sid * NPS, NPS)],
                    out.at[cid, pl.ds(sid * NPS, NPS)])

    @pl.when(sid == 0)
    def _():
        pltpu.sync_copy(deg_sh, degs.at[cid])


def kernel(features, edge_index, pseudo, weight, bias):
    f32 = jnp.float32

    feat_pad = jnp.pad(features, ((0, NPAD - N), (0, 0)))
    w2 = jnp.transpose(weight, (1, 0, 2)).reshape(F, K * F)

    pad = EPAD - E
    src2 = jnp.pad(edge_index[0], (0, pad)).reshape(ROWS, CH_E)
    dst2 = jnp.pad(edge_index[1], (0, pad)).reshape(ROWS, CH_E)
    p0 = lax.bitcast_convert_type(
        jnp.pad(pseudo[:, 0], (0, pad)).reshape(ROWS, CH_E), jnp.int32)
    p1 = lax.bitcast_convert_type(
        jnp.pad(pseudo[:, 1], (0, pad)).reshape(ROWS, CH_E), jnp.int32)
    meta = jnp.concatenate([src2, dst2, p0, p1], axis=1)
    zeros = jnp.zeros((NPS, F), f32)

    mm = pl.pallas_call(
        _mm_body,
        grid=(NB,),
        in_specs=[pl.BlockSpec((NPAD // NB, F), lambda m: (m, 0)),
                  pl.BlockSpec((F, K * F), lambda m: (0, 0))],
        out_specs=pl.BlockSpec((NPAD // NB, K * F), lambda m: (m, 0)),
        out_shape=jax.ShapeDtypeStruct((NPAD, K * F), f32),
    )
    table = mm(feat_pad, w2).reshape(NPAD * K, F)

    mesh = plsc.VectorSubcoreMesh(core_axis_name="c", subcore_axis_name="s")
    cp = pltpu.CompilerParams()
    if "needs_layout_passes" in pltpu.CompilerParams.__dataclass_fields__:
        cp = dataclasses.replace(cp, needs_layout_passes=False)
    sc = pl.kernel(
        _sc_edge_kernel,
        mesh=mesh,
        out_type=[jax.ShapeDtypeStruct((2, NAGG, F), f32),
                  jax.ShapeDtypeStruct((2, DROWS, 128), f32)],
        scratch_types=[
            pltpu.VMEM((128,), jnp.int32),
            pltpu.VMEM((CH_E,), jnp.int32),
            pltpu.VMEM((128,), jnp.int32),
            pltpu.VMEM((128, F), f32),
            pltpu.VMEM((CH_E, F), f32),
            pltpu.VMEM((DROWS, 128), f32),
            pltpu.VMEM((DROWS,), jnp.int32),
            pltpu.VMEM_SHARED((NAGG, F), f32),
            pltpu.VMEM_SHARED((DROWS, 128), f32),
        ],
        compiler_params=cp,
    )
    parts, degp = sc(table, meta, zeros)
    degf = degp.reshape(2, NAGG, 1)

    norm = pl.pallas_call(
        _norm_body,
        grid=(10,),
        in_specs=[pl.BlockSpec((2, N // 10, F), lambda i: (0, i, 0)),
                  pl.BlockSpec((2, N // 10, 1), lambda i: (0, i, 0)),
                  pl.BlockSpec((1, F), lambda i: (0, 0))],
        out_specs=pl.BlockSpec((N // 10, F), lambda i: (i, 0)),
        out_shape=jax.ShapeDtypeStruct((N, F), f32),
    )
    return norm(parts, degf, bias.reshape(1, F))
